# baseline (speedup 1.0000x reference)
import functools

import jax
import jax.numpy as jnp
import numpy as np
from jax import lax
from jax.experimental import pallas as pl
from jax.experimental.pallas import tpu as pltpu
from jax.experimental.pallas import tpu_sc as plsc

N = 100000
E = 1600000
H = 32
V = 79
C = 3
G = 256

NC, NS, LANES = 2, 16, 16
NW = NC * NS
HH = H // 2

EB = 128
TB = 12544
EPAD = TB * EB
BPS = TB // NS
CB = 28
NCHUNK = BPS // CB
NSLOT = 8
PF = 4

ACC_ROWS = 100352
TRASH = 256
RPS = ACC_ROWS // NS
NOUT = 6256
NPAD2 = NOUT * NS

DEG_ROWS = EPAD // LANES
DEG_RPW = DEG_ROWS // NW
DEG_CHUNK = 448
DEG_NCHUNK = DEG_RPW // DEG_CHUNK

NB = 5000
NBLK = N // NB

_f32 = jnp.float32
_i32 = jnp.int32


def _deg_kernel(dst16):
    mesh = plsc.VectorSubcoreMesh(core_axis_name="c", subcore_axis_name="s",
                                  num_cores=NC, num_subcores=NS)

    @functools.partial(
        pl.kernel,
        out_type=jax.ShapeDtypeStruct((NW * ACC_ROWS,), _f32),
        mesh=mesh,
        scratch_types=[
            pltpu.VMEM((ACC_ROWS,), _f32),
            pltpu.VMEM((DEG_CHUNK, LANES), _i32),
        ],
        compiler_params=pltpu.CompilerParams(needs_layout_passes=False, use_tc_tiling_on_sc=False),
    )
    def body(dst_ref, out_ref, cnt, dchunk):
        c = lax.axis_index("c")
        s = lax.axis_index("s")
        w = s * NC + c

        @pl.loop(0, ACC_ROWS // LANES, unroll=8)
        def _zero(j):
            cnt[pl.ds(j * LANES, LANES)] = jnp.zeros((LANES,), _f32)

        ones = jnp.ones((LANES,), _f32)
        base = w * DEG_RPW

        @pl.loop(0, DEG_NCHUNK)
        def _chunk(k):
            pltpu.sync_copy(
                dst_ref.at[pl.ds(base + k * DEG_CHUNK, DEG_CHUNK)], dchunk)

            @pl.loop(0, DEG_CHUNK, unroll=4)
            def _vec(j):
                plsc.addupdate_scatter(cnt, [dchunk[j]], ones)

        pltpu.sync_copy(cnt, out_ref.at[pl.ds(w * ACC_ROWS, ACC_ROWS)])

    return body(dst16)


def _agg_kernel(hs2n, srcb, dstb):
    mesh = plsc.VectorSubcoreMesh(core_axis_name="c", subcore_axis_name="s",
                                  num_cores=NC, num_subcores=NS)

    @functools.partial(
        pl.kernel,
        out_type=jax.ShapeDtypeStruct((NC, NPAD2, HH), _f32),
        mesh=mesh,
        scratch_types=[
            pltpu.VMEM_SHARED((ACC_ROWS, HH), _f32),
            pltpu.VMEM((CB, EB), _i32),
            pltpu.VMEM((CB, EB), _i32),
            pltpu.VMEM((NSLOT, EB, HH), _f32),
            pltpu.SemaphoreType.DMA((NSLOT,)),
            pltpu.SemaphoreType.DMA((NSLOT,)),
        ],
        compiler_params=pltpu.CompilerParams(needs_layout_passes=False, use_tc_tiling_on_sc=False),
    )
    def body(hs_ref, src_ref, dst_ref, out_ref, acc, sidx, didx, rows,
             gsem, ssem):
        c = lax.axis_index("c")
        s = lax.axis_index("s")

        @pl.loop(0, EB)
        def _z(j):
            rows[0, j, :] = jnp.zeros((LANES,), _f32)

        @pl.loop(0, RPS // EB)
        def _rep(j):
            pltpu.sync_copy(rows.at[0], acc.at[pl.ds(s * RPS + j * EB, EB)])

        plsc.subcore_barrier()

        def _gather(b, slot):
            pltpu.async_copy(hs_ref.at[sidx.at[b]], rows.at[slot],
                             gsem.at[slot])

        def _wait_gather(b, slot):
            pltpu.make_async_copy(hs_ref.at[sidx.at[b]], rows.at[slot],
                                  gsem.at[slot]).wait()

        def _scatter(b, slot):
            pltpu.async_copy(rows.at[slot], acc.at[didx.at[b]],
                             ssem.at[slot], add=True)

        def _wait_scatter(b, slot):
            pltpu.make_async_copy(rows.at[slot], acc.at[didx.at[b]],
                                  ssem.at[slot]).wait()

        @pl.loop(0, NCHUNK)
        def _chunk(k):
            base = s * BPS + k * CB
            pltpu.sync_copy(src_ref.at[c, pl.ds(base, CB)], sidx)
            pltpu.sync_copy(dst_ref.at[pl.ds(base, CB)], didx)

            for b in range(PF):
                _gather(b, b % NSLOT)

            @pl.loop(0, CB, unroll=2)
            def _b(b):
                slot = lax.rem(b, NSLOT)
                _wait_gather(b, slot)
                _scatter(b, slot)
                bp = b + PF

                @pl.when(bp < CB)
                def _():
                    sp = lax.rem(bp, NSLOT)

                    @pl.when(bp >= NSLOT)
                    def _():
                        _wait_scatter(bp - NSLOT, sp)

                    _gather(bp, sp)

            for b in range(CB - NSLOT, CB):
                _wait_scatter(b, b % NSLOT)

        plsc.subcore_barrier()
        pltpu.sync_copy(acc.at[pl.ds(s * NOUT, NOUT)],
                        out_ref.at[c, pl.ds(s * NOUT, NOUT)])

    return body(hs2n, srcb, dstb)


def _k1_embed(x, emb, W1):
    def body(x_ref, emb_ref, w1_ref, hs_ref):
        xb = x_ref[:, 0]
        oh = (xb[:, None] == lax.broadcasted_iota(_i32, (NB, V), 1))
        oh = oh.astype(_f32)
        emb1 = jnp.dot(emb_ref[...], w1_ref[...],
                       preferred_element_type=_f32,
                       precision=lax.Precision.HIGHEST)
        h = jnp.dot(oh, emb1, preferred_element_type=_f32)
        hs_ref[0, :, :] = h[:, :HH]
        hs_ref[1, :, :] = h[:, HH:]

    return pl.pallas_call(
        body,
        grid=(NBLK,),
        in_specs=[
            pl.BlockSpec((NB, 1), lambda i: (i, 0)),
            pl.BlockSpec((V, H), lambda i: (0, 0)),
            pl.BlockSpec((H, H), lambda i: (0, 0)),
        ],
        out_specs=pl.BlockSpec((NC, NB, HH), lambda i: (0, i, 0)),
        out_shape=jax.ShapeDtypeStruct((NC, N, HH), _f32),
    )(x, emb, W1)


def _k0_scale(deg_parts, hs1raw):
    DB = 6272
    def body(degp_ref, hsr_ref, dinv_ref, hs_ref):
        deg = jnp.sum(degp_ref[...], axis=0) + 1.0
        dv = (1.0 / jnp.sqrt(deg))[:, None]
        dinv_ref[...] = dv
        hs_ref[0, :, :] = hsr_ref[0] * dv
        hs_ref[1, :, :] = hsr_ref[1] * dv

    return pl.pallas_call(
        body,
        grid=(ACC_ROWS // DB,),
        in_specs=[
            pl.BlockSpec((NW, DB), lambda i: (0, i)),
            pl.BlockSpec((NC, DB, HH), lambda i: (0, i, 0)),
        ],
        out_specs=[
            pl.BlockSpec((DB, 1), lambda i: (i, 0)),
            pl.BlockSpec((NC, DB, HH), lambda i: (0, i, 0)),
        ],
        out_shape=[
            jax.ShapeDtypeStruct((ACC_ROWS, 1), _f32),
            jax.ShapeDtypeStruct((NC, N, HH), _f32),
        ],
    )(deg_parts, hs1raw)


def _k2_mid(agg1, hs1, dinv, W2, b1):
    def body(agg_ref, hs_ref, dinv_ref, w2_ref, b1_ref, out_ref):
        a = jnp.concatenate([agg_ref[0], agg_ref[1]], axis=1)
        sv = jnp.concatenate([hs_ref[0], hs_ref[1]], axis=1)
        dv = dinv_ref[...]
        h1 = jnp.maximum((a + sv) * dv + b1_ref[...], 0.0)
        hs2 = jnp.dot(h1, w2_ref[...], preferred_element_type=_f32) * dv
        out_ref[0, :, :] = hs2[:, :HH]
        out_ref[1, :, :] = hs2[:, HH:]

    return pl.pallas_call(
        body,
        grid=(NBLK,),
        in_specs=[
            pl.BlockSpec((NC, NB, HH), lambda i: (0, i, 0)),
            pl.BlockSpec((NC, NB, HH), lambda i: (0, i, 0)),
            pl.BlockSpec((NB, 1), lambda i: (i, 0)),
            pl.BlockSpec((H, H), lambda i: (0, 0)),
            pl.BlockSpec((1, H), lambda i: (0, 0)),
        ],
        out_specs=pl.BlockSpec((NC, NB, HH), lambda i: (0, i, 0)),
        out_shape=jax.ShapeDtypeStruct((NC, N, HH), _f32),
    )(agg1, hs1, dinv, W2, b1)


def _k3_pool(agg2, hs2, dinv, b2, batch2d, Wc, bc):
    def body(agg_ref, hs_ref, dinv_ref, b2_ref, batch_ref, wc_ref, bc_ref,
             out_ref, pool_acc, cnt_acc):
        i = pl.program_id(0)
        a = jnp.concatenate([agg_ref[0], agg_ref[1]], axis=1)
        sv = jnp.concatenate([hs_ref[0], hs_ref[1]], axis=1)
        dv = dinv_ref[...]
        h2 = jnp.maximum((a + sv) * dv + b2_ref[...], 0.0)
        oh = (batch_ref[:, 0:1] == lax.broadcasted_iota(_i32, (NB, G), 1))
        oh = oh.astype(_f32)
        part = lax.dot_general(oh, h2, (((0,), (0,)), ((), ())),
                               preferred_element_type=_f32)
        ones_col = jnp.ones((NB, 1), _f32)
        cpart = lax.dot_general(oh, ones_col, (((0,), (0,)), ((), ())),
                                preferred_element_type=_f32)

        @pl.when(i == 0)
        def _():
            pool_acc[...] = part
            cnt_acc[...] = cpart

        @pl.when(i > 0)
        def _():
            pool_acc[...] += part
            cnt_acc[...] += cpart

        @pl.when(i == NBLK - 1)
        def _():
            pooled = pool_acc[...] / jnp.maximum(cnt_acc[...], 1.0)
            out_ref[...] = jnp.dot(pooled, wc_ref[...],
                                   preferred_element_type=_f32,
                                   precision=lax.Precision.HIGHEST) + bc_ref[...]

    return pl.pallas_call(
        body,
        grid=(NBLK,),
        in_specs=[
            pl.BlockSpec((NC, NB, HH), lambda i: (0, i, 0)),
            pl.BlockSpec((NC, NB, HH), lambda i: (0, i, 0)),
            pl.BlockSpec((NB, 1), lambda i: (i, 0)),
            pl.BlockSpec((1, H), lambda i: (0, 0)),
            pl.BlockSpec((NB, 1), lambda i: (i, 0)),
            pl.BlockSpec((H, C), lambda i: (0, 0)),
            pl.BlockSpec((1, C), lambda i: (0, 0)),
        ],
        out_specs=pl.BlockSpec((G, C), lambda i: (0, 0)),
        out_shape=jax.ShapeDtypeStruct((G, C), _f32),
        scratch_shapes=[
            pltpu.VMEM((G, H), _f32),
            pltpu.VMEM((G, 1), _f32),
        ],
        compiler_params=pltpu.CompilerParams(
            dimension_semantics=("arbitrary",)),
    )(agg2, hs2, dinv, b2, batch2d, Wc, bc)


def kernel(x, edge_index, batch, emb, W1, b1, W2, b2, Wc, bc):
    x = x.astype(_i32)
    edge_index = edge_index.astype(_i32)
    batch = batch.astype(_i32)

    src = edge_index[0]
    dst = edge_index[1]
    pad = EPAD - E
    k = jnp.arange(pad, dtype=_i32)
    src_p = jnp.concatenate([src, k % np.int32(N)])
    dst_p = jnp.concatenate([dst, np.int32(N) + (k % np.int32(TRASH))])
    srcb = jnp.stack([src_p, src_p + np.int32(N)]).reshape(NC, TB, EB)
    dstb = dst_p.reshape(TB, EB)
    dst16 = dst_p.reshape(DEG_ROWS, LANES)

    deg_parts = _deg_kernel(dst16).reshape(NW, ACC_ROWS)
    hs1raw = _k1_embed(x, emb, W1)
    dinv, hs1 = _k0_scale(deg_parts, hs1raw)
    agg1 = _agg_kernel(hs1.reshape(NC * N, HH), srcb, dstb)
    hs2 = _k2_mid(agg1, hs1, dinv, W2, b1.reshape(1, H))
    agg2 = _agg_kernel(hs2.reshape(NC * N, HH), srcb, dstb)
    out = _k3_pool(agg2, hs2, dinv, b2.reshape(1, H), batch.reshape(N, 1),
                   Wc, bc.reshape(1, C))
    return out


# speedup vs baseline: 1.0374x; 1.0374x over previous
import functools

import jax
import jax.numpy as jnp
import numpy as np
from jax import lax
from jax.experimental import pallas as pl
from jax.experimental.pallas import tpu as pltpu
from jax.experimental.pallas import tpu_sc as plsc

N = 100000
E = 1600000
H = 32
V = 79
C = 3
G = 256

NC, NS, LANES = 2, 16, 16
NW = NC * NS
HH = H // 2

EB = 128
TB = 12544
EPAD = TB * EB
BPS = TB // NS
CB = 28
NCHUNK = BPS // CB
NSLOT = 8
PF = 4

ACC_ROWS = 100352
TRASH = 256
RPS = ACC_ROWS // NS
NOUT = 6256
NPAD2 = NOUT * NS

DEG_ROWS = EPAD // LANES
DEG_RPW = DEG_ROWS // NW
DEG_CHUNK = 448
DEG_NCHUNK = DEG_RPW // DEG_CHUNK

NB = 5000
NBLK = N // NB

_f32 = jnp.float32
_i32 = jnp.int32


def _deg_kernel(dst16):
    mesh = plsc.VectorSubcoreMesh(core_axis_name="c", subcore_axis_name="s",
                                  num_cores=NC, num_subcores=NS)

    @functools.partial(
        pl.kernel,
        out_type=jax.ShapeDtypeStruct((NW * ACC_ROWS,), _f32),
        mesh=mesh,
        scratch_types=[
            pltpu.VMEM((ACC_ROWS,), _f32),
            pltpu.VMEM((DEG_CHUNK, LANES), _i32),
        ],
        compiler_params=pltpu.CompilerParams(needs_layout_passes=False, use_tc_tiling_on_sc=False),
    )
    def body(dst_ref, out_ref, cnt, dchunk):
        c = lax.axis_index("c")
        s = lax.axis_index("s")
        w = s * NC + c

        @pl.loop(0, ACC_ROWS // LANES, unroll=8)
        def _zero(j):
            cnt[pl.ds(j * LANES, LANES)] = jnp.zeros((LANES,), _f32)

        ones = jnp.ones((LANES,), _f32)
        base = w * DEG_RPW

        @pl.loop(0, DEG_NCHUNK)
        def _chunk(k):
            pltpu.sync_copy(
                dst_ref.at[pl.ds(base + k * DEG_CHUNK, DEG_CHUNK)], dchunk)

            @pl.loop(0, DEG_CHUNK, unroll=4)
            def _vec(j):
                plsc.addupdate_scatter(cnt, [dchunk[j]], ones)

        pltpu.sync_copy(cnt, out_ref.at[pl.ds(w * ACC_ROWS, ACC_ROWS)])

    return body(dst16)


def _agg_kernel(hs2n, srcb, dstb):
    mesh = plsc.VectorSubcoreMesh(core_axis_name="c", subcore_axis_name="s",
                                  num_cores=NC, num_subcores=NS)

    @functools.partial(
        pl.kernel,
        out_type=jax.ShapeDtypeStruct((NC, NPAD2, HH), _f32),
        mesh=mesh,
        scratch_types=[
            pltpu.VMEM_SHARED((ACC_ROWS, HH), _f32),
            pltpu.VMEM((CB, EB), _i32),
            pltpu.VMEM((CB, EB), _i32),
            pltpu.VMEM((NSLOT, EB, HH), _f32),
            pltpu.SemaphoreType.DMA((NSLOT,)),
            pltpu.SemaphoreType.DMA((NSLOT,)),
        ],
        compiler_params=pltpu.CompilerParams(needs_layout_passes=False, use_tc_tiling_on_sc=False),
    )
    def body(hs_ref, src_ref, dst_ref, out_ref, acc, sidx, didx, rows,
             gsem, ssem):
        c = lax.axis_index("c")
        s = lax.axis_index("s")

        @pl.loop(0, EB)
        def _z(j):
            rows[0, j, :] = jnp.zeros((LANES,), _f32)

        @pl.loop(0, RPS // EB)
        def _rep(j):
            pltpu.sync_copy(rows.at[0], acc.at[pl.ds(s * RPS + j * EB, EB)])

        plsc.subcore_barrier()

        def _gather(b, slot):
            pltpu.async_copy(hs_ref.at[sidx.at[b]], rows.at[slot],
                             gsem.at[slot])

        def _wait_gather(b, slot):
            pltpu.make_async_copy(hs_ref.at[sidx.at[b]], rows.at[slot],
                                  gsem.at[slot]).wait()

        def _scatter(b, slot):
            pltpu.async_copy(rows.at[slot], acc.at[didx.at[b]],
                             ssem.at[slot], add=True)

        def _wait_scatter(b, slot):
            pltpu.make_async_copy(rows.at[slot], acc.at[didx.at[b]],
                                  ssem.at[slot]).wait()

        @pl.loop(0, NCHUNK)
        def _chunk(k):
            base = s * BPS + k * CB
            pltpu.sync_copy(src_ref.at[c, pl.ds(base, CB)], sidx)
            pltpu.sync_copy(dst_ref.at[pl.ds(base, CB)], didx)

            for b in range(PF):
                _gather(b, b % NSLOT)

            @pl.loop(0, CB, unroll=2)
            def _b(b):
                slot = lax.rem(b, NSLOT)
                _wait_gather(b, slot)
                _scatter(b, slot)
                bp = b + PF

                @pl.when(bp < CB)
                def _():
                    sp = lax.rem(bp, NSLOT)

                    @pl.when(bp >= NSLOT)
                    def _():
                        _wait_scatter(bp - NSLOT, sp)

                    _gather(bp, sp)

            for b in range(CB - NSLOT, CB):
                _wait_scatter(b, b % NSLOT)

        plsc.subcore_barrier()
        pltpu.sync_copy(acc.at[pl.ds(s * NOUT, NOUT)],
                        out_ref.at[c, pl.ds(s * NOUT, NOUT)])

    return body(hs2n, srcb, dstb)


def _k0_dinv(deg_parts):
    DB = 6272
    def body(degp_ref, dinv_ref):
        deg = jnp.sum(degp_ref[...], axis=0) + 1.0
        dinv_ref[...] = (1.0 / jnp.sqrt(deg))[:, None]

    return pl.pallas_call(
        body,
        grid=(ACC_ROWS // DB,),
        in_specs=[pl.BlockSpec((NW, DB), lambda i: (0, i))],
        out_specs=pl.BlockSpec((DB, 1), lambda i: (i, 0)),
        out_shape=jax.ShapeDtypeStruct((ACC_ROWS, 1), _f32),
    )(deg_parts)


def _k1_embed(x, dinv2, emb, W1):
    def body(x_ref, dinv_ref, emb_ref, w1_ref, hs_ref):
        xb = x_ref[:, 0]
        oh = (xb[:, None] == lax.broadcasted_iota(_i32, (NB, V), 1))
        oh = oh.astype(_f32)
        emb1 = jnp.dot(emb_ref[...], w1_ref[...],
                       preferred_element_type=_f32,
                       precision=lax.Precision.HIGHEST)
        h = jnp.dot(oh, emb1, preferred_element_type=_f32)
        hs = h * dinv_ref[...]
        hs_ref[0, :, :] = hs[:, :HH]
        hs_ref[1, :, :] = hs[:, HH:]

    return pl.pallas_call(
        body,
        grid=(NBLK,),
        in_specs=[
            pl.BlockSpec((NB, 1), lambda i: (i, 0)),
            pl.BlockSpec((NB, 1), lambda i: (i, 0)),
            pl.BlockSpec((V, H), lambda i: (0, 0)),
            pl.BlockSpec((H, H), lambda i: (0, 0)),
        ],
        out_specs=pl.BlockSpec((NC, NB, HH), lambda i: (0, i, 0)),
        out_shape=jax.ShapeDtypeStruct((NC, N, HH), _f32),
    )(x, dinv2, emb, W1)


def _k2_mid(agg1, hs1, dinv, W2, b1):
    def body(agg_ref, hs_ref, dinv_ref, w2_ref, b1_ref, out_ref):
        a = jnp.concatenate([agg_ref[0], agg_ref[1]], axis=1)
        sv = jnp.concatenate([hs_ref[0], hs_ref[1]], axis=1)
        dv = dinv_ref[...]
        h1 = jnp.maximum((a + sv) * dv + b1_ref[...], 0.0)
        hs2 = jnp.dot(h1, w2_ref[...], preferred_element_type=_f32) * dv
        out_ref[0, :, :] = hs2[:, :HH]
        out_ref[1, :, :] = hs2[:, HH:]

    return pl.pallas_call(
        body,
        grid=(NBLK,),
        in_specs=[
            pl.BlockSpec((NC, NB, HH), lambda i: (0, i, 0)),
            pl.BlockSpec((NC, NB, HH), lambda i: (0, i, 0)),
            pl.BlockSpec((NB, 1), lambda i: (i, 0)),
            pl.BlockSpec((H, H), lambda i: (0, 0)),
            pl.BlockSpec((1, H), lambda i: (0, 0)),
        ],
        out_specs=pl.BlockSpec((NC, NB, HH), lambda i: (0, i, 0)),
        out_shape=jax.ShapeDtypeStruct((NC, N, HH), _f32),
    )(agg1, hs1, dinv, W2, b1)


def _k3_pool(agg2, hs2, dinv, b2, batch2d, Wc, bc):
    def body(agg_ref, hs_ref, dinv_ref, b2_ref, batch_ref, wc_ref, bc_ref,
             out_ref, pool_acc, cnt_acc):
        i = pl.program_id(0)
        a = jnp.concatenate([agg_ref[0], agg_ref[1]], axis=1)
        sv = jnp.concatenate([hs_ref[0], hs_ref[1]], axis=1)
        dv = dinv_ref[...]
        h2 = jnp.maximum((a + sv) * dv + b2_ref[...], 0.0)
        oh = (batch_ref[:, 0:1] == lax.broadcasted_iota(_i32, (NB, G), 1))
        oh = oh.astype(_f32)
        part = lax.dot_general(oh, h2, (((0,), (0,)), ((), ())),
                               preferred_element_type=_f32)
        ones_col = jnp.ones((NB, 1), _f32)
        cpart = lax.dot_general(oh, ones_col, (((0,), (0,)), ((), ())),
                                preferred_element_type=_f32)

        @pl.when(i == 0)
        def _():
            pool_acc[...] = part
            cnt_acc[...] = cpart

        @pl.when(i > 0)
        def _():
            pool_acc[...] += part
            cnt_acc[...] += cpart

        @pl.when(i == NBLK - 1)
        def _():
            pooled = pool_acc[...] / jnp.maximum(cnt_acc[...], 1.0)
            out_ref[...] = jnp.dot(pooled, wc_ref[...],
                                   preferred_element_type=_f32,
                                   precision=lax.Precision.HIGHEST) + bc_ref[...]

    return pl.pallas_call(
        body,
        grid=(NBLK,),
        in_specs=[
            pl.BlockSpec((NC, NB, HH), lambda i: (0, i, 0)),
            pl.BlockSpec((NC, NB, HH), lambda i: (0, i, 0)),
            pl.BlockSpec((NB, 1), lambda i: (i, 0)),
            pl.BlockSpec((1, H), lambda i: (0, 0)),
            pl.BlockSpec((NB, 1), lambda i: (i, 0)),
            pl.BlockSpec((H, C), lambda i: (0, 0)),
            pl.BlockSpec((1, C), lambda i: (0, 0)),
        ],
        out_specs=pl.BlockSpec((G, C), lambda i: (0, 0)),
        out_shape=jax.ShapeDtypeStruct((G, C), _f32),
        scratch_shapes=[
            pltpu.VMEM((G, H), _f32),
            pltpu.VMEM((G, 1), _f32),
        ],
        compiler_params=pltpu.CompilerParams(
            dimension_semantics=("arbitrary",)),
    )(agg2, hs2, dinv, b2, batch2d, Wc, bc)


def kernel(x, edge_index, batch, emb, W1, b1, W2, b2, Wc, bc):
    x = x.astype(_i32)
    edge_index = edge_index.astype(_i32)
    batch = batch.astype(_i32)

    src = edge_index[0]
    dst = edge_index[1]
    pad = EPAD - E
    k = jnp.arange(pad, dtype=_i32)
    src_p = jnp.concatenate([src, k % np.int32(N)])
    dst_p = jnp.concatenate([dst, np.int32(N) + (k % np.int32(TRASH))])
    srcb = jnp.stack([src_p, src_p + np.int32(N)]).reshape(NC, TB, EB)
    dstb = dst_p.reshape(TB, EB)
    dst16 = dst_p.reshape(DEG_ROWS, LANES)

    deg_parts = _deg_kernel(dst16).reshape(NW, ACC_ROWS)
    dinv = _k0_dinv(deg_parts)
    hs1 = _k1_embed(x, dinv, emb, W1)
    agg1 = _agg_kernel(hs1.reshape(NC * N, HH), srcb, dstb)
    hs2 = _k2_mid(agg1, hs1, dinv, W2, b1.reshape(1, H))
    agg2 = _agg_kernel(hs2.reshape(NC * N, HH), srcb, dstb)
    out = _k3_pool(agg2, hs2, dinv, b2.reshape(1, H), batch.reshape(N, 1),
                   Wc, bc.reshape(1, C))
    return out


# speedup vs baseline: 1.0674x; 1.0289x over previous
import functools

import jax
import jax.numpy as jnp
import numpy as np
from jax import lax
from jax.experimental import pallas as pl
from jax.experimental.pallas import tpu as pltpu
from jax.experimental.pallas import tpu_sc as plsc

N = 100000
E = 1600000
H = 32
V = 79
C = 3
G = 256

NC, NS, LANES = 2, 16, 16
NW = NC * NS
HH = H // 2

EB = 128
TB = 12544
EPAD = TB * EB
BPS = TB // NS
CB = 49
NCHUNK = BPS // CB
NSLOT = 8
PF = 4

ACC_ROWS = 100352
TRASH = 256
RPS = ACC_ROWS // NS
NOUT = 6256
NPAD2 = NOUT * NS

DEG_ROWS = EPAD // LANES
DEG_RPW = DEG_ROWS // NW
DEG_CHUNK = 448
DEG_NCHUNK = DEG_RPW // DEG_CHUNK

NB = 5000
NBLK = N // NB

_f32 = jnp.float32
_i32 = jnp.int32


def _deg_kernel(dst16):
    mesh = plsc.VectorSubcoreMesh(core_axis_name="c", subcore_axis_name="s",
                                  num_cores=NC, num_subcores=NS)

    @functools.partial(
        pl.kernel,
        out_type=jax.ShapeDtypeStruct((NW * ACC_ROWS,), _f32),
        mesh=mesh,
        scratch_types=[
            pltpu.VMEM((ACC_ROWS,), _f32),
            pltpu.VMEM((DEG_CHUNK, LANES), _i32),
        ],
        compiler_params=pltpu.CompilerParams(needs_layout_passes=False, use_tc_tiling_on_sc=False),
    )
    def body(dst_ref, out_ref, cnt, dchunk):
        c = lax.axis_index("c")
        s = lax.axis_index("s")
        w = s * NC + c

        @pl.loop(0, ACC_ROWS // LANES, unroll=8)
        def _zero(j):
            cnt[pl.ds(j * LANES, LANES)] = jnp.zeros((LANES,), _f32)

        ones = jnp.ones((LANES,), _f32)
        base = w * DEG_RPW

        @pl.loop(0, DEG_NCHUNK)
        def _chunk(k):
            pltpu.sync_copy(
                dst_ref.at[pl.ds(base + k * DEG_CHUNK, DEG_CHUNK)], dchunk)

            @pl.loop(0, DEG_CHUNK, unroll=4)
            def _vec(j):
                plsc.addupdate_scatter(cnt, [dchunk[j]], ones)

        pltpu.sync_copy(cnt, out_ref.at[pl.ds(w * ACC_ROWS, ACC_ROWS)])

    return body(dst16)


def _agg_kernel(hs2n, srcb, dstb):
    mesh = plsc.VectorSubcoreMesh(core_axis_name="c", subcore_axis_name="s",
                                  num_cores=NC, num_subcores=NS)

    @functools.partial(
        pl.kernel,
        out_type=jax.ShapeDtypeStruct((NC, NPAD2, HH), _f32),
        mesh=mesh,
        scratch_types=[
            pltpu.VMEM_SHARED((ACC_ROWS, HH), _f32),
            pltpu.VMEM((CB, EB), _i32),
            pltpu.VMEM((CB, EB), _i32),
            pltpu.VMEM((NSLOT, EB, HH), _f32),
            pltpu.SemaphoreType.DMA((NSLOT,)),
            pltpu.SemaphoreType.DMA((NSLOT,)),
        ],
        compiler_params=pltpu.CompilerParams(needs_layout_passes=False, use_tc_tiling_on_sc=False),
    )
    def body(hs_ref, src_ref, dst_ref, out_ref, acc, sidx, didx, rows,
             gsem, ssem):
        c = lax.axis_index("c")
        s = lax.axis_index("s")

        @pl.loop(0, EB)
        def _z(j):
            rows[0, j, :] = jnp.zeros((LANES,), _f32)

        @pl.loop(0, RPS // EB)
        def _rep(j):
            pltpu.sync_copy(rows.at[0], acc.at[pl.ds(s * RPS + j * EB, EB)])

        plsc.subcore_barrier()

        def _gather(b, slot):
            pltpu.async_copy(hs_ref.at[sidx.at[b]], rows.at[slot],
                             gsem.at[slot])

        def _wait_gather(b, slot):
            pltpu.make_async_copy(hs_ref.at[sidx.at[b]], rows.at[slot],
                                  gsem.at[slot]).wait()

        def _scatter(b, slot):
            pltpu.async_copy(rows.at[slot], acc.at[didx.at[b]],
                             ssem.at[slot], add=True)

        def _wait_scatter(b, slot):
            pltpu.make_async_copy(rows.at[slot], acc.at[didx.at[b]],
                                  ssem.at[slot]).wait()

        @pl.loop(0, NCHUNK)
        def _chunk(k):
            base = s * BPS + k * CB
            pltpu.sync_copy(src_ref.at[c, pl.ds(base, CB)], sidx)
            pltpu.sync_copy(dst_ref.at[pl.ds(base, CB)], didx)

            for b in range(PF):
                _gather(b, b % NSLOT)

            @pl.loop(0, CB, unroll=2)
            def _b(b):
                slot = lax.rem(b, NSLOT)
                _wait_gather(b, slot)
                _scatter(b, slot)
                bp = b + PF

                @pl.when(bp < CB)
                def _():
                    sp = lax.rem(bp, NSLOT)

                    @pl.when(bp >= NSLOT)
                    def _():
                        _wait_scatter(bp - NSLOT, sp)

                    _gather(bp, sp)

            for b in range(CB - NSLOT, CB):
                _wait_scatter(b, b % NSLOT)

        plsc.subcore_barrier()
        pltpu.sync_copy(acc.at[pl.ds(s * NOUT, NOUT)],
                        out_ref.at[c, pl.ds(s * NOUT, NOUT)])

    return body(hs2n, srcb, dstb)


def _k0_dinv(deg_parts):
    DB = 6272
    def body(degp_ref, dinv_ref):
        deg = jnp.sum(degp_ref[...], axis=0) + 1.0
        dinv_ref[...] = (1.0 / jnp.sqrt(deg))[:, None]

    return pl.pallas_call(
        body,
        grid=(ACC_ROWS // DB,),
        in_specs=[pl.BlockSpec((NW, DB), lambda i: (0, i))],
        out_specs=pl.BlockSpec((DB, 1), lambda i: (i, 0)),
        out_shape=jax.ShapeDtypeStruct((ACC_ROWS, 1), _f32),
    )(deg_parts)


def _k1_embed(x, dinv2, emb, W1):
    def body(x_ref, dinv_ref, emb_ref, w1_ref, hs_ref):
        xb = x_ref[:, 0]
        oh = (xb[:, None] == lax.broadcasted_iota(_i32, (NB, V), 1))
        oh = oh.astype(_f32)
        emb1 = jnp.dot(emb_ref[...], w1_ref[...],
                       preferred_element_type=_f32,
                       precision=lax.Precision.HIGHEST)
        h = jnp.dot(oh, emb1, preferred_element_type=_f32)
        hs = h * dinv_ref[...]
        hs_ref[0, :, :] = hs[:, :HH]
        hs_ref[1, :, :] = hs[:, HH:]

    return pl.pallas_call(
        body,
        grid=(NBLK,),
        in_specs=[
            pl.BlockSpec((NB, 1), lambda i: (i, 0)),
            pl.BlockSpec((NB, 1), lambda i: (i, 0)),
            pl.BlockSpec((V, H), lambda i: (0, 0)),
            pl.BlockSpec((H, H), lambda i: (0, 0)),
        ],
        out_specs=pl.BlockSpec((NC, NB, HH), lambda i: (0, i, 0)),
        out_shape=jax.ShapeDtypeStruct((NC, N, HH), _f32),
    )(x, dinv2, emb, W1)


def _k2_mid(agg1, hs1, dinv, W2, b1):
    def body(agg_ref, hs_ref, dinv_ref, w2_ref, b1_ref, out_ref):
        a = jnp.concatenate([agg_ref[0], agg_ref[1]], axis=1)
        sv = jnp.concatenate([hs_ref[0], hs_ref[1]], axis=1)
        dv = dinv_ref[...]
        h1 = jnp.maximum((a + sv) * dv + b1_ref[...], 0.0)
        hs2 = jnp.dot(h1, w2_ref[...], preferred_element_type=_f32) * dv
        out_ref[0, :, :] = hs2[:, :HH]
        out_ref[1, :, :] = hs2[:, HH:]

    return pl.pallas_call(
        body,
        grid=(NBLK,),
        in_specs=[
            pl.BlockSpec((NC, NB, HH), lambda i: (0, i, 0)),
            pl.BlockSpec((NC, NB, HH), lambda i: (0, i, 0)),
            pl.BlockSpec((NB, 1), lambda i: (i, 0)),
            pl.BlockSpec((H, H), lambda i: (0, 0)),
            pl.BlockSpec((1, H), lambda i: (0, 0)),
        ],
        out_specs=pl.BlockSpec((NC, NB, HH), lambda i: (0, i, 0)),
        out_shape=jax.ShapeDtypeStruct((NC, N, HH), _f32),
    )(agg1, hs1, dinv, W2, b1)


def _k3_pool(agg2, hs2, dinv, b2, batch2d, Wc, bc):
    def body(agg_ref, hs_ref, dinv_ref, b2_ref, batch_ref, wc_ref, bc_ref,
             out_ref, pool_acc, cnt_acc):
        i = pl.program_id(0)
        a = jnp.concatenate([agg_ref[0], agg_ref[1]], axis=1)
        sv = jnp.concatenate([hs_ref[0], hs_ref[1]], axis=1)
        dv = dinv_ref[...]
        h2 = jnp.maximum((a + sv) * dv + b2_ref[...], 0.0)
        oh = (batch_ref[:, 0:1] == lax.broadcasted_iota(_i32, (NB, G), 1))
        oh = oh.astype(_f32)
        part = lax.dot_general(oh, h2, (((0,), (0,)), ((), ())),
                               preferred_element_type=_f32)
        ones_col = jnp.ones((NB, 1), _f32)
        cpart = lax.dot_general(oh, ones_col, (((0,), (0,)), ((), ())),
                                preferred_element_type=_f32)

        @pl.when(i == 0)
        def _():
            pool_acc[...] = part
            cnt_acc[...] = cpart

        @pl.when(i > 0)
        def _():
            pool_acc[...] += part
            cnt_acc[...] += cpart

        @pl.when(i == NBLK - 1)
        def _():
            pooled = pool_acc[...] / jnp.maximum(cnt_acc[...], 1.0)
            out_ref[...] = jnp.dot(pooled, wc_ref[...],
                                   preferred_element_type=_f32,
                                   precision=lax.Precision.HIGHEST) + bc_ref[...]

    return pl.pallas_call(
        body,
        grid=(NBLK,),
        in_specs=[
            pl.BlockSpec((NC, NB, HH), lambda i: (0, i, 0)),
            pl.BlockSpec((NC, NB, HH), lambda i: (0, i, 0)),
            pl.BlockSpec((NB, 1), lambda i: (i, 0)),
            pl.BlockSpec((1, H), lambda i: (0, 0)),
            pl.BlockSpec((NB, 1), lambda i: (i, 0)),
            pl.BlockSpec((H, C), lambda i: (0, 0)),
            pl.BlockSpec((1, C), lambda i: (0, 0)),
        ],
        out_specs=pl.BlockSpec((G, C), lambda i: (0, 0)),
        out_shape=jax.ShapeDtypeStruct((G, C), _f32),
        scratch_shapes=[
            pltpu.VMEM((G, H), _f32),
            pltpu.VMEM((G, 1), _f32),
        ],
        compiler_params=pltpu.CompilerParams(
            dimension_semantics=("arbitrary",)),
    )(agg2, hs2, dinv, b2, batch2d, Wc, bc)


def kernel(x, edge_index, batch, emb, W1, b1, W2, b2, Wc, bc):
    x = x.astype(_i32)
    edge_index = edge_index.astype(_i32)
    batch = batch.astype(_i32)

    src = edge_index[0]
    dst = edge_index[1]
    pad = EPAD - E
    k = jnp.arange(pad, dtype=_i32)
    src_p = jnp.concatenate([src, k % np.int32(N)])
    dst_p = jnp.concatenate([dst, np.int32(N) + (k % np.int32(TRASH))])
    srcb = jnp.stack([src_p, src_p + np.int32(N)]).reshape(NC, TB, EB)
    dstb = dst_p.reshape(TB, EB)
    dst16 = dst_p.reshape(DEG_ROWS, LANES)

    deg_parts = _deg_kernel(dst16).reshape(NW, ACC_ROWS)
    dinv = _k0_dinv(deg_parts)
    hs1 = _k1_embed(x, dinv, emb, W1)
    agg1 = _agg_kernel(hs1.reshape(NC * N, HH), srcb, dstb)
    hs2 = _k2_mid(agg1, hs1, dinv, W2, b1.reshape(1, H))
    agg2 = _agg_kernel(hs2.reshape(NC * N, HH), srcb, dstb)
    out = _k3_pool(agg2, hs2, dinv, b2.reshape(1, H), batch.reshape(N, 1),
                   Wc, bc.reshape(1, C))
    return out
